# Initial kernel scaffold; baseline (speedup 1.0000x reference)
import jax
import jax.numpy as jnp
from jax.experimental import pallas as pl


def kernel(boxes, scores):
    raise NotImplementedError("write your pallas kernel here")


# speedup vs baseline: 10.6344x; 10.6344x over previous
import functools

import jax
import jax.numpy as jnp
from jax import lax
from jax.experimental import pallas as pl
from jax.experimental.pallas import tpu as pltpu

N = 20000
LANES = 128
NMS_THRESHOLD = 0.5


def _nms_kernel(n, rows, ul0, ul1, dr0, dr1, sc, o0, o1, o2, o3, o4,
                keep, area):
    area[:] = (dr0[:] - ul0[:] + 1.0) * (dr1[:] - ul1[:] + 1.0)
    keep[:] = jnp.ones((rows, LANES), jnp.float32)
    lin = (lax.broadcasted_iota(jnp.int32, (rows, LANES), 0) * LANES
           + lax.broadcasted_iota(jnp.int32, (rows, LANES), 1))

    sub8 = lax.broadcasted_iota(jnp.int32, (8, LANES), 0)
    lane = lax.broadcasted_iota(jnp.int32, (8, LANES), 1)

    def step(i, carry):
        blk = i // (8 * LANES)
        within = i - blk * (8 * LANES)
        rsub = within // LANES
        c = within - rsub * LANES
        r8 = pl.multiple_of(blk * 8, 8)
        oh = ((sub8 == rsub) & (lane == c)).astype(jnp.float32)

        def pick(ref):
            return jnp.sum(ref[pl.ds(r8, 8), :] * oh)

        @pl.when(pick(keep) > 0.0)
        def _():
            u0 = pick(ul0)
            u1 = pick(ul1)
            d0 = pick(dr0)
            d1 = pick(dr1)
            ai = pick(area)
            iw = jnp.maximum(jnp.minimum(dr0[:], d0)
                             - jnp.maximum(ul0[:], u0) + 1.0, 0.0)
            ih = jnp.maximum(jnp.minimum(dr1[:], d1)
                             - jnp.maximum(ul1[:], u1) + 1.0, 0.0)
            inter = iw * ih
            ov = inter / (ai + area[:] - inter)
            supp = (ov >= NMS_THRESHOLD) & (lin > i)
            keep[:] = jnp.where(supp, 0.0, keep[:])

        return carry

    lax.fori_loop(0, n, step, 0)

    k = keep[:]
    o0[:] = ul0[:] * k
    o1[:] = ul1[:] * k
    o2[:] = dr0[:] * k
    o3[:] = dr1[:] * k
    o4[:] = sc[:] * k


def _run_nms(n, rows, planes):
    shp = jax.ShapeDtypeStruct((rows, LANES), jnp.float32)
    body = functools.partial(_nms_kernel, n, rows)
    return pl.pallas_call(
        body,
        out_shape=[shp] * 5,
        scratch_shapes=[pltpu.VMEM((rows, LANES), jnp.float32)] * 2,
    )(*planes)


def _nms_planar(boxes_sorted, scores_sorted, n):
    rows = (n + LANES - 1) // LANES
    rows = ((rows + 7) // 8) * 8
    npad = rows * LANES
    pad = npad - n
    b = jnp.pad(boxes_sorted, ((0, pad), (0, 0)))
    s = jnp.pad(scores_sorted, (0, pad))
    planes = [b[:, j].reshape(rows, LANES) for j in range(4)]
    planes.append(s.reshape(rows, LANES))
    outs = _run_nms(n, rows, planes)
    flat = [o.reshape(-1)[:n] for o in outs]
    return jnp.stack(flat, axis=-1)


def kernel(boxes, scores):
    order = jnp.argsort(-scores)
    b = jnp.take(boxes, order, axis=0)
    s = jnp.take(scores, order, axis=0)
    return _nms_planar(b, s, N)


# speedup vs baseline: 12.0537x; 1.1335x over previous
import functools

import jax
import jax.numpy as jnp
from jax import lax
from jax.experimental import pallas as pl
from jax.experimental.pallas import tpu as pltpu

N = 20000
LANES = 128
NMS_THRESHOLD = 0.5


def _nms_kernel(n, rows, ul0, ul1, dr0, dr1, sc, o0, o1, o2, o3, o4,
                keep, area):
    area[:] = (dr0[:] - ul0[:] + 1.0) * (dr1[:] - ul1[:] + 1.0)
    keep[:] = jnp.ones((rows, LANES), jnp.float32)
    lin = (lax.broadcasted_iota(jnp.int32, (rows, LANES), 0) * LANES
           + lax.broadcasted_iota(jnp.int32, (rows, LANES), 1))

    sub8 = lax.broadcasted_iota(jnp.int32, (8, LANES), 0)
    lane = lax.broadcasted_iota(jnp.int32, (8, LANES), 1)

    def alive_cond(i):
        return i < n

    def visit(i):
        blk = i // (8 * LANES)
        within = i - blk * (8 * LANES)
        rsub = within // LANES
        c = within - rsub * LANES
        r8 = pl.multiple_of(blk * 8, 8)
        oh = ((sub8 == rsub) & (lane == c)).astype(jnp.float32)

        def pick(ref):
            return jnp.sum(ref[pl.ds(r8, 8), :] * oh)

        u0 = pick(ul0)
        u1 = pick(ul1)
        d0 = pick(dr0)
        d1 = pick(dr1)
        ai = pick(area)
        iw = jnp.maximum(jnp.minimum(dr0[:], d0)
                         - jnp.maximum(ul0[:], u0) + 1.0, 0.0)
        ih = jnp.maximum(jnp.minimum(dr1[:], d1)
                         - jnp.maximum(ul1[:], u1) + 1.0, 0.0)
        inter = iw * ih
        ov = inter / (ai + area[:] - inter)
        later = lin > i
        supp = (ov >= NMS_THRESHOLD) & later
        newk = jnp.where(supp, 0.0, keep[:])
        keep[:] = newk
        alive = (newk > 0.0) & later
        return jnp.min(jnp.where(alive, lin, jnp.int32(n)))

    lax.while_loop(alive_cond, visit, jnp.int32(0))

    k = keep[:]
    o0[:] = ul0[:] * k
    o1[:] = ul1[:] * k
    o2[:] = dr0[:] * k
    o3[:] = dr1[:] * k
    o4[:] = sc[:] * k


def _run_nms(n, rows, planes):
    shp = jax.ShapeDtypeStruct((rows, LANES), jnp.float32)
    body = functools.partial(_nms_kernel, n, rows)
    return pl.pallas_call(
        body,
        out_shape=[shp] * 5,
        scratch_shapes=[pltpu.VMEM((rows, LANES), jnp.float32)] * 2,
    )(*planes)


def _nms_planar(boxes_sorted, scores_sorted, n):
    rows = (n + LANES - 1) // LANES
    rows = ((rows + 7) // 8) * 8
    npad = rows * LANES
    pad = npad - n
    b = jnp.pad(boxes_sorted, ((0, pad), (0, 0)))
    s = jnp.pad(scores_sorted, (0, pad))
    planes = [b[:, j].reshape(rows, LANES) for j in range(4)]
    planes.append(s.reshape(rows, LANES))
    outs = _run_nms(n, rows, planes)
    flat = [o.reshape(-1)[:n] for o in outs]
    return jnp.stack(flat, axis=-1)


def kernel(boxes, scores):
    order = jnp.argsort(-scores)
    b = jnp.take(boxes, order, axis=0)
    s = jnp.take(scores, order, axis=0)
    return _nms_planar(b, s, N)


# speedup vs baseline: 12.9926x; 1.0779x over previous
import functools

import jax
import jax.numpy as jnp
from jax import lax
from jax.experimental import pallas as pl
from jax.experimental.pallas import tpu as pltpu

N = 20000
LANES = 128
NMS_THRESHOLD = 0.5


def _nms_kernel(n, rows, ul0, ul1, dr0, dr1, sc, o0, o1, o2, o3, o4,
                keep, area):
    area[:] = (dr0[:] - ul0[:] + 1.0) * (dr1[:] - ul1[:] + 1.0)
    keep[:] = jnp.ones((rows, LANES), jnp.float32)
    lin = (lax.broadcasted_iota(jnp.int32, (rows, LANES), 0) * LANES
           + lax.broadcasted_iota(jnp.int32, (rows, LANES), 1))

    sub8 = lax.broadcasted_iota(jnp.int32, (8, LANES), 0)
    lane = lax.broadcasted_iota(jnp.int32, (8, LANES), 1)

    def alive_cond(i):
        return i < n

    def visit(i):
        blk = i // (8 * LANES)
        within = i - blk * (8 * LANES)
        rsub = within // LANES
        c = within - rsub * LANES
        r8 = pl.multiple_of(blk * 8, 8)
        oh = ((sub8 == rsub) & (lane == c)).astype(jnp.float32)

        def pick(ref):
            t = ref[pl.ds(r8, 8), :] * oh
            s = jnp.sum(t, axis=1, keepdims=True)
            return jnp.sum(s, axis=0, keepdims=True)

        u0 = pick(ul0)
        u1 = pick(ul1)
        d0 = pick(dr0)
        d1 = pick(dr1)
        ai = pick(area)
        iw = jnp.maximum(jnp.minimum(dr0[:], d0)
                         - jnp.maximum(ul0[:], u0) + 1.0, 0.0)
        ih = jnp.maximum(jnp.minimum(dr1[:], d1)
                         - jnp.maximum(ul1[:], u1) + 1.0, 0.0)
        inter = iw * ih
        ov = inter / (ai + area[:] - inter)
        later = lin > i
        supp = (ov >= NMS_THRESHOLD) & later
        newk = jnp.where(supp, 0.0, keep[:])
        keep[:] = newk
        alive = (newk > 0.0) & later
        return jnp.min(jnp.where(alive, lin, jnp.int32(n)))

    lax.while_loop(alive_cond, visit, jnp.int32(0))

    k = keep[:]
    o0[:] = ul0[:] * k
    o1[:] = ul1[:] * k
    o2[:] = dr0[:] * k
    o3[:] = dr1[:] * k
    o4[:] = sc[:] * k


def _run_nms(n, rows, planes):
    shp = jax.ShapeDtypeStruct((rows, LANES), jnp.float32)
    body = functools.partial(_nms_kernel, n, rows)
    return pl.pallas_call(
        body,
        out_shape=[shp] * 5,
        scratch_shapes=[pltpu.VMEM((rows, LANES), jnp.float32)] * 2,
    )(*planes)


def _nms_planar(boxes_sorted, scores_sorted, n):
    rows = (n + LANES - 1) // LANES
    rows = ((rows + 7) // 8) * 8
    npad = rows * LANES
    pad = npad - n
    b = jnp.pad(boxes_sorted, ((0, pad), (0, 0)))
    s = jnp.pad(scores_sorted, (0, pad))
    planes = [b[:, j].reshape(rows, LANES) for j in range(4)]
    planes.append(s.reshape(rows, LANES))
    outs = _run_nms(n, rows, planes)
    flat = [o.reshape(-1)[:n] for o in outs]
    return jnp.stack(flat, axis=-1)


def kernel(boxes, scores):
    order = jnp.argsort(-scores)
    b = jnp.take(boxes, order, axis=0)
    s = jnp.take(scores, order, axis=0)
    return _nms_planar(b, s, N)
